# baseline (speedup 1.0000x reference)
import functools

import jax
import jax.numpy as jnp
from jax import lax
from jax.experimental import pallas as pl
from jax.experimental.pallas import tpu as pltpu
from jax.experimental.pallas import tpu_sc as plsc

T = 200


def _sc_gather(table, idx):
    V, D = table.shape
    B = idx.shape[0]
    info = plsc.get_sparse_core_info()
    NC, NS = info.num_cores, info.num_subcores
    NW = NC * NS
    b_per_w = B // NW
    mesh = plsc.VectorSubcoreMesh(core_axis_name="c", subcore_axis_name="s")

    @functools.partial(
        pl.kernel,
        mesh=mesh,
        out_type=jax.ShapeDtypeStruct((B, D), jnp.float32),
        compiler_params=pltpu.CompilerParams(use_tc_tiling_on_sc=False),
        scratch_types=[
            pltpu.VMEM((b_per_w,), jnp.int32),
            pltpu.VMEM((b_per_w, D), jnp.float32),
            pltpu.SemaphoreType.DMA,
        ],
    )
    def k(table_hbm, idx_hbm, out_hbm, idx_v, rows_v, sem):
        wid = lax.axis_index("s") * NC + lax.axis_index("c")
        base = wid * b_per_w
        pltpu.sync_copy(idx_hbm.at[pl.ds(base, b_per_w)], idx_v)
        pltpu.async_copy(table_hbm.at[idx_v], rows_v, sem).wait()
        pltpu.sync_copy(rows_v, out_hbm.at[pl.ds(base, b_per_w)])

    return k(table, idx)


def _tc_expand(rows):
    D, B = rows.shape
    BBL = 128
    TT = 50

    def body(rows_ref, out_ref):
        rt = rows_ref[...]
        out_ref[...] = jnp.broadcast_to(rt[None, :, :], (TT, D, BBL))

    outT = pl.pallas_call(
        body,
        grid=(B // BBL, T // TT),
        in_specs=[pl.BlockSpec((D, BBL), lambda i, j: (0, i))],
        out_specs=pl.BlockSpec((TT, D, BBL), lambda i, j: (j, 0, i)),
        out_shape=jax.ShapeDtypeStruct((T, D, B), jnp.float32),
    )(rows)
    return jnp.transpose(outT, (2, 0, 1))


def kernel(emotion_labels, seq_len, table):
    del seq_len
    idx = emotion_labels.astype(jnp.int32)
    rows = _sc_gather(table, idx)
    return _tc_expand(rows.T)


# speedup vs baseline: 1.4116x; 1.4116x over previous
import functools

import jax
import jax.numpy as jnp
from jax import lax
from jax.experimental import pallas as pl
from jax.experimental.pallas import tpu as pltpu
from jax.experimental.pallas import tpu_sc as plsc

T = 200


def _sc_gather(table, idx):
    V, D = table.shape
    B = idx.shape[0]
    info = plsc.get_sparse_core_info()
    NC, NS = info.num_cores, info.num_subcores
    NW = NC * NS
    b_per_w = B // NW
    mesh = plsc.VectorSubcoreMesh(core_axis_name="c", subcore_axis_name="s")

    @functools.partial(
        pl.kernel,
        mesh=mesh,
        out_type=jax.ShapeDtypeStruct((B, D), jnp.float32),
        compiler_params=pltpu.CompilerParams(use_tc_tiling_on_sc=False),
        scratch_types=[
            pltpu.VMEM((b_per_w,), jnp.int32),
            pltpu.VMEM((b_per_w, D), jnp.float32),
            pltpu.SemaphoreType.DMA,
        ],
    )
    def k(table_hbm, idx_hbm, out_hbm, idx_v, rows_v, sem):
        wid = lax.axis_index("s") * NC + lax.axis_index("c")
        base = wid * b_per_w
        pltpu.sync_copy(idx_hbm.at[pl.ds(base, b_per_w)], idx_v)
        pltpu.async_copy(table_hbm.at[idx_v], rows_v, sem).wait()
        pltpu.sync_copy(rows_v, out_hbm.at[pl.ds(base, b_per_w)])

    return k(table, idx)


def _tc_expand(rows):
    D, B = rows.shape
    TT = 8

    def body(rows_ref, out_ref):
        rt = rows_ref[...]
        out_ref[...] = jnp.broadcast_to(rt[None, :, :], (TT, D, B))

    outT = pl.pallas_call(
        body,
        grid=(T // TT,),
        in_specs=[pl.BlockSpec((D, B), lambda j: (0, 0))],
        out_specs=pl.BlockSpec((TT, D, B), lambda j: (j, 0, 0)),
        out_shape=jax.ShapeDtypeStruct((T, D, B), jnp.float32),
    )(rows)
    return jnp.transpose(outT, (2, 0, 1))


def kernel(emotion_labels, seq_len, table):
    del seq_len
    idx = emotion_labels.astype(jnp.int32)
    rows = _sc_gather(table, idx)
    return _tc_expand(rows.T)


# speedup vs baseline: 1.4289x; 1.0123x over previous
import functools

import jax
import jax.numpy as jnp
from jax import lax
from jax.experimental import pallas as pl
from jax.experimental.pallas import tpu as pltpu
from jax.experimental.pallas import tpu_sc as plsc

T = 200


def _sc_gather(table, idx):
    V, D = table.shape
    B = idx.shape[0]
    info = plsc.get_sparse_core_info()
    NC, NS = info.num_cores, info.num_subcores
    NW = NC * NS
    b_per_w = B // NW
    mesh = plsc.VectorSubcoreMesh(core_axis_name="c", subcore_axis_name="s")

    @functools.partial(
        pl.kernel,
        mesh=mesh,
        out_type=jax.ShapeDtypeStruct((B, D), jnp.float32),
        compiler_params=pltpu.CompilerParams(use_tc_tiling_on_sc=False),
        scratch_types=[
            pltpu.VMEM((b_per_w,), jnp.int32),
            pltpu.VMEM((b_per_w, D), jnp.float32),
            pltpu.SemaphoreType.DMA,
        ],
    )
    def k(table_hbm, idx_hbm, out_hbm, idx_v, rows_v, sem):
        wid = lax.axis_index("s") * NC + lax.axis_index("c")
        base = wid * b_per_w
        pltpu.sync_copy(idx_hbm.at[pl.ds(base, b_per_w)], idx_v)
        pltpu.async_copy(table_hbm.at[idx_v], rows_v, sem).wait()
        pltpu.sync_copy(rows_v, out_hbm.at[pl.ds(base, b_per_w)])

    return k(table, idx)


def _tc_expand(rows):
    D, B = rows.shape
    TT = 4

    def body(rows_ref, out_ref):
        rt = rows_ref[...]
        out_ref[...] = jnp.broadcast_to(rt[None, :, :], (TT, D, B))

    outT = pl.pallas_call(
        body,
        grid=(T // TT,),
        in_specs=[pl.BlockSpec((D, B), lambda j: (0, 0))],
        out_specs=pl.BlockSpec((TT, D, B), lambda j: (j, 0, 0)),
        out_shape=jax.ShapeDtypeStruct((T, D, B), jnp.float32),
    )(rows)
    return jnp.transpose(outT, (2, 0, 1))


def kernel(emotion_labels, seq_len, table):
    del seq_len
    idx = emotion_labels.astype(jnp.int32)
    rows = _sc_gather(table, idx)
    return _tc_expand(rows.T)
